# baseline (speedup 1.0000x reference)
import functools
import math

import jax
import jax.numpy as jnp
from jax import lax
from jax.experimental import pallas as pl
from jax.experimental.pallas import tpu as pltpu
from jax.experimental.pallas import tpu_sc as plsc

CUTOFF = 0.8

E = 6400000
EF = 1600000

NC = 2
NS = 16
NW = NC * NS
PER_W = EF // NW

DCHUNK = 10000
NDCHUNK = PER_W // DCHUNK
VCHUNK = 10000
NVCHUNK = PER_W // VCHUNK
L = 16


def _switch_chunk(d_v, sw_v, mf_v):
    half = math.pi / (2.0 * CUTOFF)

    def body(j, _):
        d = d_v[pl.ds(j * L, L)]
        y = d * half
        t = y * y
        c = 1.0 + t * (-0.5 + t * (1.0 / 24.0 + t * (-1.0 / 720.0
                                                     + t * (1.0 / 40320.0))))
        s = c * c
        m = d < CUTOFF
        sw_v[pl.ds(j * L, L)] = jnp.where(m, s, 0.0)
        mf_v[pl.ds(j * L, L)] = jnp.where(m, 1.0, 0.0)
        return 0

    lax.fori_loop(0, DCHUNK // L, body, 0)


def _sc_dist_body(dist_hbm, idx_hbm, od_hbm, osw_hbm, omf_hbm,
                  idx_v0, idx_v1, d_v0, d_v1, sw_v0, sw_v1, mf_v0, mf_v1,
                  sg, sw):
    idx_v = [idx_v0, idx_v1]
    d_v = [d_v0, d_v1]
    sw_v = [sw_v0, sw_v1]
    mf_v = [mf_v0, mf_v1]
    wid = lax.axis_index("s") * NC + lax.axis_index("c")
    base_w = wid * PER_W
    cp_g = [None, None]
    cp_w = [None, None]

    def emit_chunk(pb, pbase):
        _switch_chunk(d_v[pb], sw_v[pb], mf_v[pb])
        return [
            pltpu.async_copy(d_v[pb], od_hbm.at[pl.ds(pbase, DCHUNK)], sw),
            pltpu.async_copy(sw_v[pb], osw_hbm.at[pl.ds(pbase, DCHUNK)], sw),
            pltpu.async_copy(mf_v[pb], omf_hbm.at[pl.ds(pbase, DCHUNK)], sw),
        ]

    for c in range(NDCHUNK):
        b = c & 1
        if cp_w[b] is not None:
            for cp in cp_w[b]:
                cp.wait()
        pltpu.sync_copy(idx_hbm.at[pl.ds(base_w + c * DCHUNK, DCHUNK)],
                        idx_v[b])
        cp_g[b] = pltpu.async_copy(dist_hbm.at[idx_v[b]], d_v[b], sg)
        if c > 0:
            pb = 1 - b
            cp_g[pb].wait()
            cp_w[pb] = emit_chunk(pb, base_w + (c - 1) * DCHUNK)
    lb = (NDCHUNK - 1) & 1
    cp_g[lb].wait()
    for cp in emit_chunk(lb, base_w + (NDCHUNK - 1) * DCHUNK):
        cp.wait()
    if NDCHUNK > 1:
        for cp in cp_w[1 - lb]:
            cp.wait()


_sc_dist = functools.partial(
    pl.kernel,
    mesh=plsc.VectorSubcoreMesh(core_axis_name="c", subcore_axis_name="s"),
    out_type=[jax.ShapeDtypeStruct((EF,), jnp.float32)] * 3,
    scratch_types=(
        [pltpu.VMEM((DCHUNK,), jnp.int32)] * 2
        + [pltpu.VMEM((DCHUNK,), jnp.float32)] * 6
        + [pltpu.SemaphoreType.DMA, pltpu.SemaphoreType.DMA]
    ),
)(_sc_dist_body)


EH = EF // 2
PER_H = EH // NW
HCHUNK = 5000
NHCHUNK = PER_H // HCHUNK


def _make_sc_vec_half(base0):
    def body(vx_hbm, vy_hbm, vz_hbm, idx_hbm, ox_hbm, oy_hbm, oz_hbm,
             idx_v0, idx_v1, x_v0, x_v1, y_v0, y_v1, z_v0, z_v1, sg, sw):
        idx_v = [idx_v0, idx_v1]
        x_v = [x_v0, x_v1]
        y_v = [y_v0, y_v1]
        z_v = [z_v0, z_v1]
        wid = lax.axis_index("s") * NC + lax.axis_index("c")
        base_w = wid * PER_H
        cp_g = [None, None]
        cp_w = [None, None]
        for c in range(NHCHUNK):
            b = c & 1
            if cp_w[b] is not None:
                for cp in cp_w[b]:
                    cp.wait()
            pltpu.sync_copy(
                idx_hbm.at[pl.ds(base0 + base_w + c * HCHUNK, HCHUNK)],
                idx_v[b])
            cp_g[b] = [
                pltpu.async_copy(vx_hbm.at[idx_v[b]], x_v[b], sg),
                pltpu.async_copy(vy_hbm.at[idx_v[b]], y_v[b], sg),
                pltpu.async_copy(vz_hbm.at[idx_v[b]], z_v[b], sg),
            ]
            if c > 0:
                pb = 1 - b
                pbase = base_w + (c - 1) * HCHUNK
                for cp in cp_g[pb]:
                    cp.wait()
                cp_w[pb] = [
                    pltpu.async_copy(x_v[pb], ox_hbm.at[pl.ds(pbase, HCHUNK)], sw),
                    pltpu.async_copy(y_v[pb], oy_hbm.at[pl.ds(pbase, HCHUNK)], sw),
                    pltpu.async_copy(z_v[pb], oz_hbm.at[pl.ds(pbase, HCHUNK)], sw),
                ]
        lb = (NHCHUNK - 1) & 1
        lbase = base_w + (NHCHUNK - 1) * HCHUNK
        for cp in cp_g[lb]:
            cp.wait()
        pltpu.sync_copy(x_v[lb], ox_hbm.at[pl.ds(lbase, HCHUNK)])
        pltpu.sync_copy(y_v[lb], oy_hbm.at[pl.ds(lbase, HCHUNK)])
        pltpu.sync_copy(z_v[lb], oz_hbm.at[pl.ds(lbase, HCHUNK)])
        if NHCHUNK > 1:
            for cp in cp_w[1 - lb]:
                cp.wait()

    return functools.partial(
        pl.kernel,
        mesh=plsc.VectorSubcoreMesh(core_axis_name="c", subcore_axis_name="s"),
        out_type=[jax.ShapeDtypeStruct((EH,), jnp.float32)] * 3,
        scratch_types=(
            [pltpu.VMEM((HCHUNK,), jnp.int32)] * 2
            + [pltpu.VMEM((HCHUNK,), jnp.float32)] * 6
            + [pltpu.SemaphoreType.DMA, pltpu.SemaphoreType.DMA]
        ),
    )(body)


_sc_vec_lo = _make_sc_vec_half(0)
_sc_vec_hi = _make_sc_vec_half(EH)


def _interleave_half(xf, yf, zf):
    ci = lax.broadcasted_iota(jnp.int32, (EH, 3), 1)
    bx = lax.broadcast_in_dim(xf, (EH, 3), (0,))
    by = lax.broadcast_in_dim(yf, (EH, 3), (0,))
    bz = lax.broadcast_in_dim(zf, (EH, 3), (0,))
    return jnp.where(ci == 0, bx, jnp.where(ci == 1, by, bz))


def kernel(vec, distances, filter_indices):
    dist_f, switch, maskf = _sc_dist(distances, filter_indices)
    vx, vy, vz = vec[:, 0], vec[:, 1], vec[:, 2]
    idx2 = lax.optimization_barrier((filter_indices, dist_f))[0]
    xf1, yf1, zf1 = _sc_vec_lo(vx, vy, vz, idx2)
    idx3 = lax.optimization_barrier((filter_indices, xf1))[0]
    xf2, yf2, zf2 = _sc_vec_hi(vx, vy, vz, idx3)
    vh1 = _interleave_half(xf1, yf1, zf1)
    vh2 = _interleave_half(xf2, yf2, zf2)
    vec_f = jnp.zeros((EF, 3), jnp.float32)
    vec_f = lax.dynamic_update_slice(vec_f, vh1, (0, 0))
    vec_f = lax.dynamic_update_slice(vec_f, vh2, (EH, 0))
    return vec_f, dist_f, switch, maskf.astype(jnp.bool_)


# speedup vs baseline: 1.0117x; 1.0117x over previous
import functools
import math

import jax
import jax.numpy as jnp
from jax import lax
from jax.experimental import pallas as pl
from jax.experimental.pallas import tpu as pltpu
from jax.experimental.pallas import tpu_sc as plsc

CUTOFF = 0.8

E = 6400000
EF = 1600000

NC = 2
NS = 16
NW = NC * NS
PER_W = EF // NW

DCHUNK = 10000
NDCHUNK = PER_W // DCHUNK
VCHUNK = 10000
NVCHUNK = PER_W // VCHUNK
L = 16


def _switch_chunk(d_v, sw_v, mf_v):
    half = math.pi / (2.0 * CUTOFF)

    def body(j, _):
        d = d_v[pl.ds(j * L, L)]
        y = d * half
        t = y * y
        c = 1.0 + t * (-0.5 + t * (1.0 / 24.0 + t * (-1.0 / 720.0
                                                     + t * (1.0 / 40320.0))))
        s = c * c
        m = d < CUTOFF
        sw_v[pl.ds(j * L, L)] = jnp.where(m, s, 0.0)
        mf_v[pl.ds(j * L, L)] = jnp.where(m, 1.0, 0.0)
        return 0

    lax.fori_loop(0, DCHUNK // L, body, 0)


def _sc_dist_body(dist_hbm, idx_hbm, od_hbm, osw_hbm, omf_hbm,
                  idx_v0, idx_v1, d_v0, d_v1, sw_v0, sw_v1, mf_v0, mf_v1,
                  sg, sw):
    idx_v = [idx_v0, idx_v1]
    d_v = [d_v0, d_v1]
    sw_v = [sw_v0, sw_v1]
    mf_v = [mf_v0, mf_v1]
    wid = lax.axis_index("s") * NC + lax.axis_index("c")
    base_w = wid * PER_W
    cp_g = [None, None]
    cp_w = [None, None]

    def emit_chunk(pb, pbase):
        _switch_chunk(d_v[pb], sw_v[pb], mf_v[pb])
        return [
            pltpu.async_copy(d_v[pb], od_hbm.at[pl.ds(pbase, DCHUNK)], sw),
            pltpu.async_copy(sw_v[pb], osw_hbm.at[pl.ds(pbase, DCHUNK)], sw),
            pltpu.async_copy(mf_v[pb], omf_hbm.at[pl.ds(pbase, DCHUNK)], sw),
        ]

    for c in range(NDCHUNK):
        b = c & 1
        if cp_w[b] is not None:
            for cp in cp_w[b]:
                cp.wait()
        pltpu.sync_copy(idx_hbm.at[pl.ds(base_w + c * DCHUNK, DCHUNK)],
                        idx_v[b])
        cp_g[b] = pltpu.async_copy(dist_hbm.at[idx_v[b]], d_v[b], sg)
        if c > 0:
            pb = 1 - b
            cp_g[pb].wait()
            cp_w[pb] = emit_chunk(pb, base_w + (c - 1) * DCHUNK)
    lb = (NDCHUNK - 1) & 1
    cp_g[lb].wait()
    for cp in emit_chunk(lb, base_w + (NDCHUNK - 1) * DCHUNK):
        cp.wait()
    if NDCHUNK > 1:
        for cp in cp_w[1 - lb]:
            cp.wait()


_sc_dist = functools.partial(
    pl.kernel,
    mesh=plsc.VectorSubcoreMesh(core_axis_name="c", subcore_axis_name="s"),
    out_type=[jax.ShapeDtypeStruct((EF,), jnp.float32)] * 3,
    scratch_types=(
        [pltpu.VMEM((DCHUNK,), jnp.int32)] * 2
        + [pltpu.VMEM((DCHUNK,), jnp.float32)] * 6
        + [pltpu.SemaphoreType.DMA, pltpu.SemaphoreType.DMA]
    ),
)(_sc_dist_body)


def _sc_vec_body(vx_hbm, vy_hbm, vz_hbm, idx_hbm, ox_hbm, oy_hbm, oz_hbm,
                 idx_v0, idx_v1, x_v0, x_v1, y_v0, y_v1, z_v0, z_v1, sg, sw):
    idx_v = [idx_v0, idx_v1]
    x_v = [x_v0, x_v1]
    y_v = [y_v0, y_v1]
    z_v = [z_v0, z_v1]
    wid = lax.axis_index("s") * NC + lax.axis_index("c")
    base_w = wid * PER_W
    cp_g = [None, None]
    cp_w = [None, None]
    for c in range(NVCHUNK):
        b = c & 1
        if cp_w[b] is not None:
            for cp in cp_w[b]:
                cp.wait()
        pltpu.sync_copy(idx_hbm.at[pl.ds(base_w + c * VCHUNK, VCHUNK)],
                        idx_v[b])
        cp_g[b] = [
            pltpu.async_copy(vx_hbm.at[idx_v[b]], x_v[b], sg),
            pltpu.async_copy(vy_hbm.at[idx_v[b]], y_v[b], sg),
            pltpu.async_copy(vz_hbm.at[idx_v[b]], z_v[b], sg),
        ]
        if c > 0:
            pb = 1 - b
            pbase = base_w + (c - 1) * VCHUNK
            for cp in cp_g[pb]:
                cp.wait()
            cp_w[pb] = [
                pltpu.async_copy(x_v[pb], ox_hbm.at[pl.ds(pbase, VCHUNK)], sw),
                pltpu.async_copy(y_v[pb], oy_hbm.at[pl.ds(pbase, VCHUNK)], sw),
                pltpu.async_copy(z_v[pb], oz_hbm.at[pl.ds(pbase, VCHUNK)], sw),
            ]
    lb = (NVCHUNK - 1) & 1
    lbase = base_w + (NVCHUNK - 1) * VCHUNK
    for cp in cp_g[lb]:
        cp.wait()
    pltpu.sync_copy(x_v[lb], ox_hbm.at[pl.ds(lbase, VCHUNK)])
    pltpu.sync_copy(y_v[lb], oy_hbm.at[pl.ds(lbase, VCHUNK)])
    pltpu.sync_copy(z_v[lb], oz_hbm.at[pl.ds(lbase, VCHUNK)])
    if NVCHUNK > 1:
        for cp in cp_w[1 - lb]:
            cp.wait()


_sc_vec = functools.partial(
    pl.kernel,
    mesh=plsc.VectorSubcoreMesh(core_axis_name="c", subcore_axis_name="s"),
    out_type=[jax.ShapeDtypeStruct((EF,), jnp.float32)] * 3,
    scratch_types=(
        [pltpu.VMEM((VCHUNK,), jnp.int32)] * 2
        + [pltpu.VMEM((VCHUNK,), jnp.float32)] * 6
        + [pltpu.SemaphoreType.DMA, pltpu.SemaphoreType.DMA]
    ),
)(_sc_vec_body)


def kernel(vec, distances, filter_indices):
    dist_f, switch, maskf = _sc_dist(distances, filter_indices)
    vt = vec.T
    vx, vy, vz = vt[0], vt[1], vt[2]
    idx2 = lax.optimization_barrier((filter_indices, dist_f))[0]
    xf, yf, zf = _sc_vec(vx, vy, vz, idx2)
    ci = lax.broadcasted_iota(jnp.int32, (EF, 3), 1)
    bx = lax.broadcast_in_dim(xf, (EF, 3), (0,))
    by = lax.broadcast_in_dim(yf, (EF, 3), (0,))
    bz = lax.broadcast_in_dim(zf, (EF, 3), (0,))
    vec_f = jnp.where(ci == 0, bx, jnp.where(ci == 1, by, bz))
    return vec_f, dist_f, switch, maskf.astype(jnp.bool_)
